# Initial kernel scaffold; baseline (speedup 1.0000x reference)
import jax
import jax.numpy as jnp
from jax.experimental import pallas as pl


def kernel(coords, bonds, encoded, t, answer, W0, b0, W1, b1, W2, b2, W3, b3):
    raise NotImplementedError("write your pallas kernel here")


# speedup vs baseline: 3.2846x; 3.2846x over previous
import functools

import jax
import jax.numpy as jnp
from jax import lax
from jax.experimental import pallas as pl
from jax.experimental.pallas import tpu as pltpu
from jax.experimental.pallas import tpu_sc as plsc

LEAKY = 0.001
NC = 2
NS = 16
CHUNK = 128


def _leaky(x):
    return jnp.maximum(x, LEAKY * x)


def _proj_body(enc_ref, wa_ref, wb_ref, c_ref, p_ref, q_ref):
    e = enc_ref[...]
    p_ref[...] = jnp.dot(e, wa_ref[...], preferred_element_type=jnp.float32) + c_ref[...]
    q_ref[...] = jnp.dot(e, wb_ref[...], preferred_element_type=jnp.float32)


def _edge_body(s_ref, dr_ref, w1_ref, b1_ref, w2_ref,
               b2_ref, w3_ref, b3_ref, wdl_ref, u0_ref, u1_ref):
    dr = dr_ref[...]
    dl2 = jnp.sum(dr * dr, axis=1, keepdims=True)
    dl = jnp.sqrt(jnp.maximum(dl2, 1e-12))
    h = s_ref[...] + dl * wdl_ref[...]
    x = _leaky(h)
    x = _leaky(jnp.dot(x, w1_ref[...], preferred_element_type=jnp.float32) + b1_ref[...])
    x = _leaky(jnp.dot(x, w2_ref[...], preferred_element_type=jnp.float32) + b2_ref[...])
    d8 = jnp.dot(x, w3_ref[...], preferred_element_type=jnp.float32) + b3_ref[...]
    dh = dr / dl
    u0_ref[...] = -0.5 * d8[:, 0:1] * dh
    u1_ref[...] = 0.5 * d8[:, 1:2] * dh


def _combine_body(ans_ref, part_ref, out_ref):
    p = part_ref[...]
    out_ref[...] = ans_ref[...] + jnp.sum(p, axis=0)


def kernel(coords, bonds, encoded, t, answer, W0, b0, W1, b1, W2, b2, W3, b3):
    N, T, _ = coords.shape
    E = bonds.shape[0]
    D = encoded.shape[1]
    NT = NC * NS
    ept = pl.cdiv(E, NT * CHUNK) * CHUNK
    e_pad = ept * NT
    nch = ept // CHUNK
    nch_slow = 24
    nch_fast = 2 * nch - nch_slow

    f32 = jnp.float32

    i0 = bonds[:, 0]
    i1 = bonds[:, 1]
    pad = e_pad - E
    i0p = jnp.pad(i0, (0, pad)).reshape(e_pad // CHUNK, CHUNK)
    i1p = jnp.pad(i1, (0, pad)).reshape(e_pad // CHUNK, CHUNK)
    coords4 = jnp.pad(coords.reshape(N, 3 * T), ((0, 0), (0, 1))).reshape(N * 4)
    ans4 = jnp.pad(answer.reshape(N, 3 * T), ((0, 0), (0, 1)))
    zeros4 = jnp.zeros((N, 4), f32)
    W0a = W0[:D]
    W0b = W0[D:2 * D]
    cvec = (b0 + t[0] * W0[2 * D]).reshape(1, D)
    wdl = W0[2 * D + 1].reshape(1, D)
    W3p = jnp.pad(W3, ((0, 0), (0, 8 - W3.shape[1])))
    b3p = jnp.pad(b3, (0, 8 - b3.shape[0])).reshape(1, 8)
    b1r = b1.reshape(1, D)
    b2r = b2.reshape(1, D)

    nb = 1000
    grid_a = N // nb
    P, Q = pl.pallas_call(
        _proj_body,
        grid=(grid_a,),
        in_specs=[
            pl.BlockSpec((nb, D), lambda i: (i, 0)),
            pl.BlockSpec((D, D), lambda i: (0, 0)),
            pl.BlockSpec((D, D), lambda i: (0, 0)),
            pl.BlockSpec((1, D), lambda i: (0, 0)),
        ],
        out_specs=[
            pl.BlockSpec((nb, D), lambda i: (i, 0)),
            pl.BlockSpec((nb, D), lambda i: (i, 0)),
        ],
        out_shape=[
            jax.ShapeDtypeStruct((N, D), f32),
            jax.ShapeDtypeStruct((N, D), f32),
        ],
    )(encoded, W0a, W0b, cvec)

    mesh = plsc.VectorSubcoreMesh(core_axis_name="c", subcore_axis_name="s",
                                  num_cores=NC, num_subcores=NS)

    @functools.partial(
        pl.kernel,
        out_type=jax.ShapeDtypeStruct((e_pad, D), f32),
        mesh=mesh,
        scratch_types=[
            pltpu.VMEM((nch_fast, CHUNK), jnp.int32),
            pltpu.VMEM((nch_fast, CHUNK), jnp.int32),
            pltpu.VMEM((2, CHUNK, D), f32),
            pltpu.VMEM((2, CHUNK, D), f32),
            pltpu.SemaphoreType.DMA,
            pltpu.SemaphoreType.DMA,
            pltpu.SemaphoreType.DMA,
            pltpu.SemaphoreType.DMA,
        ],
        compiler_params=pltpu.CompilerParams(needs_layout_passes=False),
    )
    def gather_kernel(p_hbm, q_hbm, i0_hbm, i1_hbm,
                      s_out,
                      i0m, i1m, gp, gq,
                      gsem0, gsem1, wsem0, wsem1):
        cid = lax.axis_index("c")
        sid = lax.axis_index("s")
        row0 = jnp.where(cid == 1, sid * nch_slow,
                         NS * nch_slow + sid * nch_fast)
        my_pairs = jnp.where(cid == 1, nch_slow // 2, nch_fast // 2)
        pltpu.sync_copy(i0_hbm.at[pl.ds(row0, nch_fast)], i0m)
        pltpu.sync_copy(i1_hbm.at[pl.ds(row0, nch_fast)], i1m)
        gsems = (gsem0, gsem1)
        wsems = (wsem0, wsem1)

        def start_chunk(j, s):
            return [
                pltpu.async_copy(p_hbm.at[i0m.at[j]], gp.at[s], gsems[s]),
                pltpu.async_copy(q_hbm.at[i1m.at[j]], gq.at[s], gsems[s]),
            ]

        def finish_chunk(j, s, gathers):
            base = pl.multiple_of((row0 + j) * CHUNK, CHUNK)
            for g in gathers:
                g.wait()

            def add_row(r, carry):
                for k in range(D // 16):
                    sl = pl.ds(k * 16, 16)
                    gp[s, r, sl] = gp[s, r, sl] + gq[s, r, sl]
                return carry

            lax.fori_loop(0, CHUNK, add_row, 0)
            return [
                pltpu.async_copy(gp.at[s], s_out.at[pl.ds(base, CHUNK)], wsems[s]),
            ]

        def body(t, carry):
            j0 = t * 2
            j1 = j0 + 1
            g0 = start_chunk(j0, 0)
            g1 = start_chunk(j1, 1)
            w0 = finish_chunk(j0, 0, g0)
            w1 = finish_chunk(j1, 1, g1)
            for w in w0 + w1:
                w.wait()
            return carry

        lax.fori_loop(0, my_pairs, body, 0)


    @functools.partial(
        pl.kernel,
        out_type=jax.ShapeDtypeStruct((e_pad, 4), f32),
        mesh=mesh,
        scratch_types=[
            pltpu.VMEM((nch, CHUNK), jnp.int32),
            pltpu.VMEM((nch, CHUNK), jnp.int32),
            pltpu.VMEM((CHUNK, 4), f32),
            pltpu.VMEM((N * 4,), f32),
        ],
        compiler_params=pltpu.CompilerParams(needs_layout_passes=False),
    )
    def dr_kernel(c4_hbm, i0_hbm, i1_hbm, dr_out, i0m, i1m, drs, cv):
        wid = lax.axis_index("c") * NS + lax.axis_index("s")
        tbase = wid * ept
        pltpu.sync_copy(i0_hbm.at[pl.ds(wid * nch, nch)], i0m)
        pltpu.sync_copy(i1_hbm.at[pl.ds(wid * nch, nch)], i1m)
        pltpu.sync_copy(c4_hbm, cv)

        def body(j, carry):
            base = pl.multiple_of(tbase + j * CHUNK, CHUNK)
            for g in range(CHUNK // 16):
                r0 = i0m[j, pl.ds(g * 16, 16)] * 4
                r1 = i1m[j, pl.ds(g * 16, 16)] * 4
                rows = jnp.arange(16, dtype=jnp.int32) + (g * 16)
                for c in range(4):
                    v0 = plsc.load_gather(cv, [r0 + c])
                    v1 = plsc.load_gather(cv, [r1 + c])
                    plsc.store_scatter(drs, [rows, jnp.full((16,), c, jnp.int32)],
                                       v0 - v1)
            pltpu.sync_copy(drs, dr_out.at[pl.ds(base, CHUNK)])
            return carry

        lax.fori_loop(0, nch, body, 0)

    S = gather_kernel(P, Q, i0p, i1p)
    DR = dr_kernel(coords4, i0p, i1p)

    eb = 4096
    grid_c = e_pad // eb
    U0, U1 = pl.pallas_call(
        _edge_body,
        grid=(grid_c,),
        in_specs=[
            pl.BlockSpec((eb, D), lambda i: (i, 0)),
            pl.BlockSpec((eb, 4), lambda i: (i, 0)),
            pl.BlockSpec((D, D), lambda i: (0, 0)),
            pl.BlockSpec((1, D), lambda i: (0, 0)),
            pl.BlockSpec((D, D), lambda i: (0, 0)),
            pl.BlockSpec((1, D), lambda i: (0, 0)),
            pl.BlockSpec((D, 8), lambda i: (0, 0)),
            pl.BlockSpec((1, 8), lambda i: (0, 0)),
            pl.BlockSpec((1, D), lambda i: (0, 0)),
        ],
        out_specs=[
            pl.BlockSpec((eb, 4), lambda i: (i, 0)),
            pl.BlockSpec((eb, 4), lambda i: (i, 0)),
        ],
        out_shape=[
            jax.ShapeDtypeStruct((e_pad, 4), f32),
            jax.ShapeDtypeStruct((e_pad, 4), f32),
        ],
    )(S, DR, W1, b1r, W2, b2r, W3p, b3p, wdl)

    z4f = zeros4.reshape(N * 4)

    @functools.partial(
        pl.kernel,
        out_type=jax.ShapeDtypeStruct((NT, N * 4), f32),
        mesh=mesh,
        scratch_types=[
            pltpu.VMEM((CHUNK,), jnp.int32),
            pltpu.VMEM((CHUNK,), jnp.int32),
            pltpu.VMEM((CHUNK, 4), f32),
            pltpu.VMEM((CHUNK, 4), f32),
            pltpu.VMEM((N * 4,), f32),
        ],
        compiler_params=pltpu.CompilerParams(needs_layout_passes=False),
    )
    def scatter_kernel(u0_hbm, u1_hbm, i0_hbm, i1_hbm, z4_hbm, part_hbm,
                       idx0v, idx1v, u0v, u1v, accl):
        cid = lax.axis_index("c")
        sid = lax.axis_index("s")
        wid = cid * NS + sid
        tbase = wid * ept
        pltpu.sync_copy(z4_hbm, accl)

        def scatter_chunk(idxv, uv):
            for g in range(CHUNK // 16):
                eidx = idxv[pl.ds(g * 16, 16)] * 4
                rows = jnp.arange(16, dtype=jnp.int32) + (g * 16)
                for c in range(3):
                    vals = plsc.load_gather(uv, [rows, jnp.full((16,), c, jnp.int32)])
                    plsc.addupdate_scatter(accl, [eidx + c], vals)

        def body(j, carry):
            base = pl.multiple_of(tbase + j * CHUNK, CHUNK)
            pltpu.sync_copy(i0_hbm.at[wid * nch + j], idx0v)
            pltpu.sync_copy(i1_hbm.at[wid * nch + j], idx1v)
            pltpu.sync_copy(u0_hbm.at[pl.ds(base, CHUNK)], u0v)
            pltpu.sync_copy(u1_hbm.at[pl.ds(base, CHUNK)], u1v)
            scatter_chunk(idx0v, u0v)
            scatter_chunk(idx1v, u1v)
            return carry

        lax.fori_loop(0, nch, body, 0)
        pltpu.sync_copy(accl, part_hbm.at[wid])

    part = scatter_kernel(U0, U1, i0p, i1p, z4f)

    ans_out4 = pl.pallas_call(
        _combine_body,
        grid=(1,),
        in_specs=[
            pl.BlockSpec((1, N * 4), lambda i: (0, 0)),
            pl.BlockSpec((NT, N * 4), lambda i: (0, 0)),
        ],
        out_specs=pl.BlockSpec((1, N * 4), lambda i: (0, 0)),
        out_shape=jax.ShapeDtypeStruct((1, N * 4), f32),
    )(ans4.reshape(1, N * 4), part)

    return ans_out4.reshape(N, 4)[:, :3].reshape(N, T, 3)
